# Initial kernel scaffold; baseline (speedup 1.0000x reference)
import jax
import jax.numpy as jnp
from jax.experimental import pallas as pl


def kernel(xyz_pred, xyz_ref, feat_pred, feat_ref_coord, feat_ref, Wv_w, Wv_b, Wo_w, Wo_b, Wout_w, Wout_b):
    raise NotImplementedError("write your pallas kernel here")


# speedup vs baseline: 3.6057x; 3.6057x over previous
import functools
import math

import jax
import jax.numpy as jnp
from jax import lax
from jax.experimental import pallas as pl
from jax.experimental.pallas import tpu as pltpu
from jax.experimental.pallas import tpu_sc as plsc

N = 16384
C = 256
K = 8
QB_KNN = 64
QB_ATT = 128
RB_V = 512
CHK = 128
NW = 32


def _knn_body(q_ref, r_ref, idx_ref):
    q = q_ref[...]
    qx = q[:, 0:1]
    qy = q[:, 1:2]
    qz = q[:, 2:3]
    rx = r_ref[0:1, :]
    ry = r_ref[1:2, :]
    rz = r_ref[2:3, :]
    qsq = qx * qx + qy * qy + qz * qz
    rsq = rx * rx + ry * ry + rz * rz

    def _rb(x):
        return x.astype(jnp.bfloat16).astype(jnp.float32)

    cross = _rb(qx) * _rb(rx) + _rb(qy) * _rb(ry) + _rb(qz) * _rb(rz)
    d = (qsq - 2.0 * cross) + rsq

    iota = lax.broadcasted_iota(jnp.int32, (QB_KNN, N), 1)
    cols = []
    for _ in range(K):
        m = jnp.min(d, axis=1, keepdims=True)
        cand = jnp.where(d == m, iota, N)
        a = jnp.min(cand, axis=1, keepdims=True)
        cols.append(a)
        d = jnp.where(iota == a, jnp.inf, d)
    idx_ref[...] = jnp.concatenate(cols, axis=1)


def _knn(xyz_pred_p, xyz_refT_p):
    grid = N // QB_KNN
    return pl.pallas_call(
        _knn_body,
        grid=(grid,),
        in_specs=[
            pl.BlockSpec((QB_KNN, 8), lambda i: (i, 0)),
            pl.BlockSpec((8, N), lambda i: (0, 0)),
        ],
        out_specs=pl.BlockSpec((QB_KNN, K), lambda i: (i, 0)),
        out_shape=jax.ShapeDtypeStruct((N, K), jnp.int32),
    )(xyz_pred_p, xyz_refT_p)


def _vraw_body(f_ref, w_ref, b_ref, o_ref):
    o_ref[...] = (
        jnp.dot(f_ref[...], w_ref[...], preferred_element_type=jnp.float32)
        + b_ref[...]
    )


def _vraw(feat_ref, WvT, Wv_b2):
    grid = N // RB_V
    return pl.pallas_call(
        _vraw_body,
        grid=(grid,),
        in_specs=[
            pl.BlockSpec((RB_V, C), lambda i: (i, 0)),
            pl.BlockSpec((C, C), lambda i: (0, 0)),
            pl.BlockSpec((1, C), lambda i: (0, 0)),
        ],
        out_specs=pl.BlockSpec((RB_V, C), lambda i: (i, 0)),
        out_shape=jax.ShapeDtypeStruct((N, C), jnp.float32),
    )(feat_ref, WvT, Wv_b2)


def _make_gather():
    B = N * K
    b_per_w = B // NW
    n_chunks = b_per_w // CHK
    mesh = plsc.VectorSubcoreMesh(core_axis_name="c", subcore_axis_name="s")

    @functools.partial(
        pl.kernel,
        mesh=mesh,
        out_type=[
            jax.ShapeDtypeStruct((B, C), jnp.float32),
            jax.ShapeDtypeStruct((B, C), jnp.float32),
        ],
        scratch_types=[
            pltpu.VMEM((b_per_w,), jnp.int32),
            pltpu.VMEM((CHK, C), jnp.float32),
            pltpu.VMEM((CHK, C), jnp.float32),
            pltpu.SemaphoreType.DMA,
            pltpu.SemaphoreType.DMA,
        ],
    )
    def gather(tab1, tab2, idx_hbm, out1, out2, idx_v, rows1, rows2, s1, s2):
        wid = lax.axis_index("s") * 2 + lax.axis_index("c")
        base = wid * b_per_w
        pltpu.sync_copy(idx_hbm.at[pl.ds(base, b_per_w)], idx_v)

        def body(c, carry):
            off = c * CHK
            idx_c = idx_v.at[pl.ds(off, CHK)]
            cp1 = pltpu.async_copy(tab1.at[idx_c], rows1, s1)
            cp2 = pltpu.async_copy(tab2.at[idx_c], rows2, s2)
            cp1.wait()
            cp2.wait()
            pltpu.sync_copy(rows1, out1.at[pl.ds(base + off, CHK)])
            pltpu.sync_copy(rows2, out2.at[pl.ds(base + off, CHK)])
            return carry

        lax.fori_loop(0, n_chunks, body, 0)

    return gather


def _attn_body(q_ref, kg_ref, vg_ref, woT_ref, bo_ref, wout_ref, bout_ref,
               o_ref):
    q = q_ref[...]
    kg = kg_ref[...]
    vg = vg_ref[...]
    scale = 1.0 / math.sqrt(C)
    logit_cols = []
    for j in range(K):
        kj = kg[:, j * C:(j + 1) * C]
        logit_cols.append(jnp.sum(q * kj, axis=1, keepdims=True) * scale)
    logits = jnp.concatenate(logit_cols, axis=1)
    m = jnp.max(logits, axis=1, keepdims=True)
    e = jnp.exp(logits - m)
    w = e / jnp.sum(e, axis=1, keepdims=True)
    pred = jnp.zeros_like(q)
    for j in range(K):
        pred = pred + w[:, j:j + 1] * vg[:, j * C:(j + 1) * C]
    pred = (
        jnp.dot(pred, woT_ref[...], preferred_element_type=jnp.float32)
        + bo_ref[...]
    )
    o_ref[...] = (
        jnp.dot(pred, wout_ref[...], preferred_element_type=jnp.float32)
        + bout_ref[...]
    )


def _attn(feat_pred, kg2, vg2, WoT, Wo_b2, Wout_w, Wout_b2):
    grid = N // QB_ATT
    return pl.pallas_call(
        _attn_body,
        grid=(grid,),
        in_specs=[
            pl.BlockSpec((QB_ATT, C), lambda i: (i, 0)),
            pl.BlockSpec((QB_ATT, K * C), lambda i: (i, 0)),
            pl.BlockSpec((QB_ATT, K * C), lambda i: (i, 0)),
            pl.BlockSpec((C, C), lambda i: (0, 0)),
            pl.BlockSpec((1, C), lambda i: (0, 0)),
            pl.BlockSpec((C, C), lambda i: (0, 0)),
            pl.BlockSpec((1, C), lambda i: (0, 0)),
        ],
        out_specs=pl.BlockSpec((QB_ATT, C), lambda i: (i, 0)),
        out_shape=jax.ShapeDtypeStruct((N, C), jnp.float32),
    )(feat_pred, kg2, vg2, WoT, Wo_b2, Wout_w, Wout_b2)


def kernel(xyz_pred, xyz_ref, feat_pred, feat_ref_coord, feat_ref,
           Wv_w, Wv_b, Wo_w, Wo_b, Wout_w, Wout_b):
    xyz_pred_p = jnp.pad(xyz_pred, ((0, 0), (0, 5)))
    xyz_refT_p = jnp.pad(xyz_ref.T, ((0, 5), (0, 0)))

    idx = _knn(xyz_pred_p, xyz_refT_p)
    v_raw = _vraw(feat_ref, Wv_w.T, Wv_b[None, :])

    idx_flat = idx.reshape(N * K)
    kg, vg = _make_gather()(feat_ref_coord, v_raw, idx_flat)
    kg2 = kg.reshape(N, K * C)
    vg2 = vg.reshape(N, K * C)

    out = _attn(feat_pred, kg2, vg2, Wo_w.T, Wo_b[None, :],
                Wout_w, Wout_b[None, :])
    return out


# speedup vs baseline: 4.6616x; 1.2928x over previous
import functools
import math

import jax
import jax.numpy as jnp
from jax import lax
from jax.experimental import pallas as pl
from jax.experimental.pallas import tpu as pltpu
from jax.experimental.pallas import tpu_sc as plsc

N = 16384
C = 256
K = 8
QB_KNN = 64
BIN = 128
QB_ATT = 128
RB_V = 512
CHK = 128
NW = 32


def _knn_body(q_ref, r_ref, idx_ref):
    q = q_ref[...]
    qx = q[:, 0:1]
    qy = q[:, 1:2]
    qz = q[:, 2:3]
    rx = r_ref[0:1, :]
    ry = r_ref[1:2, :]
    rz = r_ref[2:3, :]
    qsq = qx * qx + qy * qy + qz * qz
    rsq = rx * rx + ry * ry + rz * rz

    qm = (-2.0 * q).astype(jnp.bfloat16)
    rm = r_ref[...].astype(jnp.bfloat16)
    neg2cross = jax.lax.dot_general(
        qm, rm, (((1,), (0,)), ((), ())),
        preferred_element_type=jnp.float32)
    d = (qsq + neg2cross) + rsq

    NB = 128
    BI = N // NB
    d3 = d.reshape(QB_KNN, BI, NB)
    binmin = jnp.min(d3, axis=1)

    iota_b = lax.broadcasted_iota(jnp.int32, (QB_KNN, NB), 1)
    bm = binmin
    bcols = []
    for _ in range(K):
        m = jnp.min(bm, axis=1, keepdims=True)
        cand = jnp.where(bm == m, iota_b, NB)
        b = jnp.min(cand, axis=1, keepdims=True)
        bcols.append(b)
        bm = jnp.where(iota_b == b, jnp.inf, bm)
    bsel = jnp.concatenate(bcols, axis=1)

    bidx = jnp.broadcast_to(bsel[:, None, :], (QB_KNN, BI, K))
    cval = jnp.take_along_axis(d3, bidx, axis=2)
    cval = cval.reshape(QB_KNN, BI * K)
    bsel_pad = jnp.pad(bsel, ((0, 0), (0, NB - K)))
    iota_p = lax.broadcasted_iota(jnp.int32, (QB_KNN, BI * K), 1)
    bsel_rep = jnp.take_along_axis(bsel_pad, iota_p & (K - 1), axis=1)
    gidx = (iota_p >> 3) * NB + bsel_rep

    cols = []
    for _ in range(K):
        m = jnp.min(cval, axis=1, keepdims=True)
        cand = jnp.where(cval == m, gidx, N)
        a = jnp.min(cand, axis=1, keepdims=True)
        cols.append(a)
        cval = jnp.where(gidx == a, jnp.inf, cval)
    idx_ref[...] = jnp.concatenate(cols, axis=1)


def _knn(xyz_pred_p, xyz_refT_p):
    grid = N // QB_KNN
    return pl.pallas_call(
        _knn_body,
        grid=(grid,),
        in_specs=[
            pl.BlockSpec((QB_KNN, 8), lambda i: (i, 0)),
            pl.BlockSpec((8, N), lambda i: (0, 0)),
        ],
        out_specs=pl.BlockSpec((QB_KNN, K), lambda i: (i, 0)),
        out_shape=jax.ShapeDtypeStruct((N, K), jnp.int32),
    )(xyz_pred_p, xyz_refT_p)


def _vraw_body(f_ref, w_ref, b_ref, o_ref):
    o_ref[...] = (
        jnp.dot(f_ref[...], w_ref[...], preferred_element_type=jnp.float32)
        + b_ref[...]
    )


def _vraw(feat_ref, WvT, Wv_b2):
    grid = N // RB_V
    return pl.pallas_call(
        _vraw_body,
        grid=(grid,),
        in_specs=[
            pl.BlockSpec((RB_V, C), lambda i: (i, 0)),
            pl.BlockSpec((C, C), lambda i: (0, 0)),
            pl.BlockSpec((1, C), lambda i: (0, 0)),
        ],
        out_specs=pl.BlockSpec((RB_V, C), lambda i: (i, 0)),
        out_shape=jax.ShapeDtypeStruct((N, C), jnp.float32),
    )(feat_ref, WvT, Wv_b2)


def _make_gather():
    B = N * K
    b_per_w = B // NW
    n_chunks = b_per_w // CHK
    mesh = plsc.VectorSubcoreMesh(core_axis_name="c", subcore_axis_name="s")

    @functools.partial(
        pl.kernel,
        mesh=mesh,
        out_type=[
            jax.ShapeDtypeStruct((B, C), jnp.float32),
            jax.ShapeDtypeStruct((B, C), jnp.float32),
        ],
        scratch_types=[
            pltpu.VMEM((b_per_w,), jnp.int32),
            pltpu.VMEM((CHK, C), jnp.float32),
            pltpu.VMEM((CHK, C), jnp.float32),
            pltpu.SemaphoreType.DMA,
            pltpu.SemaphoreType.DMA,
        ],
    )
    def gather(tab1, tab2, idx_hbm, out1, out2, idx_v, rows1, rows2, s1, s2):
        wid = lax.axis_index("s") * 2 + lax.axis_index("c")
        base = wid * b_per_w
        pltpu.sync_copy(idx_hbm.at[pl.ds(base, b_per_w)], idx_v)

        def body(c, carry):
            off = c * CHK
            idx_c = idx_v.at[pl.ds(off, CHK)]
            cp1 = pltpu.async_copy(tab1.at[idx_c], rows1, s1)
            cp2 = pltpu.async_copy(tab2.at[idx_c], rows2, s2)
            cp1.wait()
            cp2.wait()
            pltpu.sync_copy(rows1, out1.at[pl.ds(base + off, CHK)])
            pltpu.sync_copy(rows2, out2.at[pl.ds(base + off, CHK)])
            return carry

        lax.fori_loop(0, n_chunks, body, 0)

    return gather


def _attn_body(q_ref, kg_ref, vg_ref, woT_ref, bo_ref, wout_ref, bout_ref,
               o_ref):
    q = q_ref[...]
    kg = kg_ref[...]
    vg = vg_ref[...]
    scale = 1.0 / math.sqrt(C)
    logit_cols = []
    for j in range(K):
        kj = kg[:, j * C:(j + 1) * C]
        logit_cols.append(jnp.sum(q * kj, axis=1, keepdims=True) * scale)
    logits = jnp.concatenate(logit_cols, axis=1)
    m = jnp.max(logits, axis=1, keepdims=True)
    e = jnp.exp(logits - m)
    w = e / jnp.sum(e, axis=1, keepdims=True)
    pred = jnp.zeros_like(q)
    for j in range(K):
        pred = pred + w[:, j:j + 1] * vg[:, j * C:(j + 1) * C]
    pred = (
        jnp.dot(pred, woT_ref[...], preferred_element_type=jnp.float32)
        + bo_ref[...]
    )
    o_ref[...] = (
        jnp.dot(pred, wout_ref[...], preferred_element_type=jnp.float32)
        + bout_ref[...]
    )


def _attn(feat_pred, kg2, vg2, WoT, Wo_b2, Wout_w, Wout_b2):
    grid = N // QB_ATT
    return pl.pallas_call(
        _attn_body,
        grid=(grid,),
        in_specs=[
            pl.BlockSpec((QB_ATT, C), lambda i: (i, 0)),
            pl.BlockSpec((QB_ATT, K * C), lambda i: (i, 0)),
            pl.BlockSpec((QB_ATT, K * C), lambda i: (i, 0)),
            pl.BlockSpec((C, C), lambda i: (0, 0)),
            pl.BlockSpec((1, C), lambda i: (0, 0)),
            pl.BlockSpec((C, C), lambda i: (0, 0)),
            pl.BlockSpec((1, C), lambda i: (0, 0)),
        ],
        out_specs=pl.BlockSpec((QB_ATT, C), lambda i: (i, 0)),
        out_shape=jax.ShapeDtypeStruct((N, C), jnp.float32),
    )(feat_pred, kg2, vg2, WoT, Wo_b2, Wout_w, Wout_b2)


def kernel(xyz_pred, xyz_ref, feat_pred, feat_ref_coord, feat_ref,
           Wv_w, Wv_b, Wo_w, Wo_b, Wout_w, Wout_b):
    xyz_pred_p = jnp.pad(xyz_pred, ((0, 0), (0, 5)))
    xyz_refT_p = jnp.pad(xyz_ref.T, ((0, 5), (0, 0)))

    idx = _knn(xyz_pred_p, xyz_refT_p)
    v_raw = _vraw(feat_ref, Wv_w.T, Wv_b[None, :])

    idx_flat = idx.reshape(N * K)
    kg, vg = _make_gather()(feat_ref_coord, v_raw, idx_flat)
    kg2 = kg.reshape(N, K * C)
    vg2 = vg.reshape(N, K * C)

    out = _attn(feat_pred, kg2, vg2, Wo_w.T, Wo_b[None, :],
                Wout_w, Wout_b[None, :])
    return out


# speedup vs baseline: 6.5411x; 1.4032x over previous
import functools
import math

import jax
import jax.numpy as jnp
from jax import lax
from jax.experimental import pallas as pl
from jax.experimental.pallas import tpu as pltpu
from jax.experimental.pallas import tpu_sc as plsc

N = 16384
C = 256
K = 8
QB_KNN = 256
BIN = 128
QB_ATT = 128
RB_V = 512
CHK = 128
NW = 32


def _knn_body(q_ref, r_ref, idx_ref):
    q = q_ref[...]
    qx = q[:, 0:1]
    qy = q[:, 1:2]
    qz = q[:, 2:3]
    rx = r_ref[0:1, :]
    ry = r_ref[1:2, :]
    rz = r_ref[2:3, :]
    qsq = qx * qx + qy * qy + qz * qz
    rsq = rx * rx + ry * ry + rz * rz

    qm = (-2.0 * q).astype(jnp.bfloat16)
    rm = r_ref[...].astype(jnp.bfloat16)
    neg2cross = jax.lax.dot_general(
        qm, rm, (((1,), (0,)), ((), ())),
        preferred_element_type=jnp.float32)
    d = (qsq + neg2cross) + rsq

    NB = 128
    BI = N // NB
    d3 = d.reshape(QB_KNN, BI, NB)
    binmin = jnp.min(d3, axis=1)

    iota_b = lax.broadcasted_iota(jnp.int32, (QB_KNN, NB), 1)
    bm = binmin
    bcols = []
    for _ in range(K):
        m = jnp.min(bm, axis=1, keepdims=True)
        cand = jnp.where(bm == m, iota_b, NB)
        b = jnp.min(cand, axis=1, keepdims=True)
        bcols.append(b)
        bm = jnp.where(iota_b == b, jnp.inf, bm)
    bsel = jnp.concatenate(bcols, axis=1)

    bidx = jnp.broadcast_to(bsel[:, None, :], (QB_KNN, BI, K))
    cval = jnp.take_along_axis(d3, bidx, axis=2)
    cval = cval.reshape(QB_KNN, BI * K)
    bsel_pad = jnp.pad(bsel, ((0, 0), (0, NB - K)))
    iota_p = lax.broadcasted_iota(jnp.int32, (QB_KNN, BI * K), 1)
    bsel_rep = jnp.take_along_axis(bsel_pad, iota_p & (K - 1), axis=1)
    gidx = (iota_p >> 3) * NB + bsel_rep

    cols = []
    for _ in range(K):
        m = jnp.min(cval, axis=1, keepdims=True)
        cand = jnp.where(cval == m, gidx, N)
        a = jnp.min(cand, axis=1, keepdims=True)
        cols.append(a)
        cval = jnp.where(gidx == a, jnp.inf, cval)
    idx_ref[...] = jnp.concatenate(cols, axis=1)


def _knn(xyz_pred_p, xyz_refT_p):
    nq = xyz_pred_p.shape[0]
    grid = nq // QB_KNN
    return pl.pallas_call(
        _knn_body,
        grid=(grid,),
        in_specs=[
            pl.BlockSpec((QB_KNN, 8), lambda i: (i, 0)),
            pl.BlockSpec((8, N), lambda i: (0, 0)),
        ],
        out_specs=pl.BlockSpec((QB_KNN, K), lambda i: (i, 0)),
        out_shape=jax.ShapeDtypeStruct((nq, K), jnp.int32),
    )(xyz_pred_p, xyz_refT_p)


def _vraw_body(f_ref, w_ref, b_ref, o_ref):
    o_ref[...] = (
        jnp.dot(f_ref[...], w_ref[...], preferred_element_type=jnp.float32)
        + b_ref[...]
    )


def _vraw(feat_ref, WvT, Wv_b2):
    grid = N // RB_V
    return pl.pallas_call(
        _vraw_body,
        grid=(grid,),
        in_specs=[
            pl.BlockSpec((RB_V, C), lambda i: (i, 0)),
            pl.BlockSpec((C, C), lambda i: (0, 0)),
            pl.BlockSpec((1, C), lambda i: (0, 0)),
        ],
        out_specs=pl.BlockSpec((RB_V, C), lambda i: (i, 0)),
        out_shape=jax.ShapeDtypeStruct((N, C), jnp.float32),
    )(feat_ref, WvT, Wv_b2)


def _make_gather(nq):
    B = nq * K
    b_per_w = B // NW
    n_chunks = b_per_w // CHK
    mesh = plsc.VectorSubcoreMesh(core_axis_name="c", subcore_axis_name="s")

    @functools.partial(
        pl.kernel,
        mesh=mesh,
        out_type=[
            jax.ShapeDtypeStruct((B, C), jnp.float32),
            jax.ShapeDtypeStruct((B, C), jnp.float32),
        ],
        scratch_types=[
            pltpu.VMEM((b_per_w,), jnp.int32),
            pltpu.VMEM((CHK, C), jnp.float32),
            pltpu.VMEM((CHK, C), jnp.float32),
            pltpu.SemaphoreType.DMA,
            pltpu.SemaphoreType.DMA,
        ],
    )
    def gather(tab1, tab2, idx_hbm, out1, out2, idx_v, rows1, rows2, s1, s2):
        wid = lax.axis_index("s") * 2 + lax.axis_index("c")
        base = wid * b_per_w
        pltpu.sync_copy(idx_hbm.at[pl.ds(base, b_per_w)], idx_v)

        def body(c, carry):
            off = c * CHK
            idx_c = idx_v.at[pl.ds(off, CHK)]
            cp1 = pltpu.async_copy(tab1.at[idx_c], rows1, s1)
            cp2 = pltpu.async_copy(tab2.at[idx_c], rows2, s2)
            cp1.wait()
            cp2.wait()
            pltpu.sync_copy(rows1, out1.at[pl.ds(base + off, CHK)])
            pltpu.sync_copy(rows2, out2.at[pl.ds(base + off, CHK)])
            return carry

        lax.fori_loop(0, n_chunks, body, 0)

    return gather


def _attn_body(q_ref, kg_ref, vg_ref, woT_ref, bo_ref, wout_ref, bout_ref,
               o_ref):
    q = q_ref[...]
    kg = kg_ref[...]
    vg = vg_ref[...]
    scale = 1.0 / math.sqrt(C)
    logit_cols = []
    for j in range(K):
        kj = kg[:, j * C:(j + 1) * C]
        logit_cols.append(jnp.sum(q * kj, axis=1, keepdims=True) * scale)
    logits = jnp.concatenate(logit_cols, axis=1)
    m = jnp.max(logits, axis=1, keepdims=True)
    e = jnp.exp(logits - m)
    w = e / jnp.sum(e, axis=1, keepdims=True)
    pred = jnp.zeros_like(q)
    for j in range(K):
        pred = pred + w[:, j:j + 1] * vg[:, j * C:(j + 1) * C]
    pred = (
        jnp.dot(pred, woT_ref[...], preferred_element_type=jnp.float32)
        + bo_ref[...]
    )
    o_ref[...] = (
        jnp.dot(pred, wout_ref[...], preferred_element_type=jnp.float32)
        + bout_ref[...]
    )


def _attn(feat_pred, kg2, vg2, WoT, Wo_b2, Wout_w, Wout_b2):
    nq = feat_pred.shape[0]
    grid = nq // QB_ATT
    return pl.pallas_call(
        _attn_body,
        grid=(grid,),
        in_specs=[
            pl.BlockSpec((QB_ATT, C), lambda i: (i, 0)),
            pl.BlockSpec((QB_ATT, K * C), lambda i: (i, 0)),
            pl.BlockSpec((QB_ATT, K * C), lambda i: (i, 0)),
            pl.BlockSpec((C, C), lambda i: (0, 0)),
            pl.BlockSpec((1, C), lambda i: (0, 0)),
            pl.BlockSpec((C, C), lambda i: (0, 0)),
            pl.BlockSpec((1, C), lambda i: (0, 0)),
        ],
        out_specs=pl.BlockSpec((QB_ATT, C), lambda i: (i, 0)),
        out_shape=jax.ShapeDtypeStruct((nq, C), jnp.float32),
    )(feat_pred, kg2, vg2, WoT, Wo_b2, Wout_w, Wout_b2)


def kernel(xyz_pred, xyz_ref, feat_pred, feat_ref_coord, feat_ref,
           Wv_w, Wv_b, Wo_w, Wo_b, Wout_w, Wout_b):
    xyz_pred_p = jnp.pad(xyz_pred, ((0, 0), (0, 5)))
    xyz_refT_p = jnp.pad(xyz_ref.T, ((0, 5), (0, 0)))

    v_raw = _vraw(feat_ref, Wv_w.T, Wv_b[None, :])

    H = N // 2
    gather = _make_gather(H)
    WoT = Wo_w.T
    Wo_b2 = Wo_b[None, :]
    Wout_b2 = Wout_b[None, :]

    outs = []
    kgs = []
    for h in range(2):
        sl = slice(h * H, (h + 1) * H)
        idx = _knn(xyz_pred_p[sl], xyz_refT_p)
        kg, vg = gather(feat_ref_coord, v_raw, idx.reshape(H * K))
        kgs.append((kg.reshape(H, K * C), vg.reshape(H, K * C)))
    for h in range(2):
        sl = slice(h * H, (h + 1) * H)
        kg2, vg2 = kgs[h]
        outs.append(_attn(feat_pred[sl], kg2, vg2, WoT, Wo_b2,
                          Wout_w, Wout_b2))
    return jnp.concatenate(outs, axis=0)
